# Initial kernel scaffold; baseline (speedup 1.0000x reference)
import functools

import jax
import jax.numpy as jnp
from jax import lax
from jax.experimental import pallas as pl
from jax.experimental.pallas import tpu as pltpu
from jax.experimental.pallas import tpu_sc as plsc

VOCAB = 100000
DIM = 128
SEQ = 200
BATCH = 1024

NC = 2
NS = 16
NW = NC * NS

BLOCK = 128
N_FLAT = SEQ * BATCH
NUM_BLOCKS = N_FLAT // BLOCK
BLOCKS_PER_W = NUM_BLOCKS // NW
BLOCKS_PER_L = BATCH // BLOCK
NBUF = 5
NGROUPS = BLOCKS_PER_W // NBUF
LANES = 16
NVEC = DIM // LANES


def _emb_body(src_hbm, table_hbm, pe_hbm, out_hbm, idx_v, pe_v, rows_v, *sems):
    gsems = sems[:NBUF]
    ssems = sems[NBUF:]
    c = lax.axis_index("c")
    s = lax.axis_index("s")
    w = s * NC + c

    pltpu.sync_copy(src_hbm.at[pl.ds(w * BLOCKS_PER_W, BLOCKS_PER_W)], idx_v)
    pltpu.sync_copy(pe_hbm, pe_v)

    first_blk = w * BLOCKS_PER_W

    def gather_descr(j, b):
        return pltpu.make_async_copy(
            table_hbm.at[idx_v.at[j]], rows_v.at[b], gsems[b])

    def scatter_descr(j, b):
        out_base = (first_blk + j) * BLOCK
        return pltpu.make_async_copy(
            rows_v.at[b], out_hbm.at[pl.ds(out_base, BLOCK)], ssems[b])

    def group(g, carry):
        base = g * NBUF
        for b in range(NBUF):
            j = base + b

            @pl.when(g > 0)
            def _wait_prev_scatter():
                scatter_descr(j - NBUF, b).wait()

            gather_descr(j, b).start()

        for b in range(NBUF):
            j = base + b
            gather_descr(j, b).wait()

            l = (first_blk + j) // BLOCKS_PER_L
            pe_vecs = [pe_v[l, pl.ds(d * LANES, LANES)] for d in range(NVEC)]

            def row_body(r, acc, b=b, pe_vecs=pe_vecs):
                for d in range(NVEC):
                    plsc.addupdate(
                        rows_v.at[b, r, pl.ds(d * LANES, LANES)], pe_vecs[d])
                return acc

            lax.fori_loop(0, BLOCK, row_body, 0)
            scatter_descr(j, b).start()
        return carry

    lax.fori_loop(0, NGROUPS, group, 0)
    for b in range(NBUF):
        scatter_descr((NGROUPS - 1) * NBUF + b, b).wait()


def _positional_encoding_const():
    positions = jnp.arange(SEQ, dtype=jnp.float32)[:, None]
    i = jnp.arange(DIM, dtype=jnp.float32)
    div_term = 1.0 / jnp.power(10000.0, (2.0 * i) / DIM)
    angles = positions * div_term[None, :]
    even_mask = (jnp.arange(DIM) % 2 == 0)
    return jnp.where(even_mask[None, :], jnp.sin(angles), jnp.cos(angles))


@functools.partial(
    pl.kernel,
    out_type=jax.ShapeDtypeStruct((N_FLAT, DIM), jnp.float32),
    mesh=plsc.VectorSubcoreMesh(core_axis_name="c", subcore_axis_name="s"),
    scratch_types=[
        pltpu.VMEM((BLOCKS_PER_W, BLOCK), jnp.int32),
        pltpu.VMEM((SEQ, DIM), jnp.float32),
        pltpu.VMEM((NBUF, BLOCK, DIM), jnp.float32),
    ] + [pltpu.SemaphoreType.DMA] * (2 * NBUF),
)
def _emb_sc_kernel(src_hbm, table_hbm, pe_hbm, out_hbm, *scratch):
    _emb_body(src_hbm, table_hbm, pe_hbm, out_hbm, *scratch)


def kernel(src, table):
    idx = src.reshape(NUM_BLOCKS, BLOCK)
    pe = _positional_encoding_const()
    out = _emb_sc_kernel(idx, table, pe)
    return out.reshape(SEQ, BATCH, DIM)


# speedup vs baseline: 7.3477x; 7.3477x over previous
import functools

import jax
import jax.numpy as jnp
from jax import lax
from jax.experimental import pallas as pl
from jax.experimental.pallas import tpu as pltpu
from jax.experimental.pallas import tpu_sc as plsc

VOCAB = 100000
DIM = 128
SEQ = 200
BATCH = 1024

NC = 2
NS = 16
NW = NC * NS

BLOCK = 128
N_FLAT = SEQ * BATCH
NUM_BLOCKS = N_FLAT // BLOCK
BLOCKS_PER_W = NUM_BLOCKS // NW
BLOCKS_PER_L = BATCH // BLOCK
NBUF = 5
NGROUPS = BLOCKS_PER_W // NBUF
LANES = 16
NVEC = DIM // LANES


def _emb_body(src_hbm, table_hbm, pe_hbm, out_hbm, idx_v, pe_v, rows_v, *sems):
    gsems = sems[:NBUF]
    ssems = sems[NBUF:]
    c = lax.axis_index("c")
    s = lax.axis_index("s")
    w = s * NC + c

    pltpu.sync_copy(src_hbm.at[w], idx_v)
    pltpu.sync_copy(pe_hbm, pe_v)

    first_blk = w * BLOCKS_PER_W

    def gather_descr(j, b):
        return pltpu.make_async_copy(
            table_hbm.at[idx_v.at[j]], rows_v.at[b], gsems[b])

    def scatter_descr(j, b):
        out_base = (first_blk + j) * BLOCK
        return pltpu.make_async_copy(
            rows_v.at[b], out_hbm.at[pl.ds(out_base, BLOCK)], ssems[b])

    def group(g, carry):
        base = g * NBUF
        for b in range(NBUF):
            j = base + b

            @pl.when(g > 0)
            def _wait_prev_scatter():
                scatter_descr(j - NBUF, b).wait()

            gather_descr(j, b).start()

        for b in range(NBUF):
            j = base + b
            gather_descr(j, b).wait()

            l = (first_blk + j) // BLOCKS_PER_L
            pe_vecs = [pe_v[l, pl.ds(d * LANES, LANES)] for d in range(NVEC)]

            def row_body(r, acc, b=b, pe_vecs=pe_vecs):
                for d in range(NVEC):
                    plsc.addupdate(
                        rows_v.at[b, r, pl.ds(d * LANES, LANES)], pe_vecs[d])
                return acc

            lax.fori_loop(0, BLOCK, row_body, 0)
            scatter_descr(j, b).start()
        return carry

    lax.fori_loop(0, NGROUPS, group, 0)
    for b in range(NBUF):
        scatter_descr((NGROUPS - 1) * NBUF + b, b).wait()


def _positional_encoding_const():
    positions = jnp.arange(SEQ, dtype=jnp.float32)[:, None]
    i = jnp.arange(DIM, dtype=jnp.float32)
    div_term = 1.0 / jnp.power(10000.0, (2.0 * i) / DIM)
    angles = positions * div_term[None, :]
    even_mask = (jnp.arange(DIM) % 2 == 0)
    return jnp.where(even_mask[None, :], jnp.sin(angles), jnp.cos(angles))


@functools.partial(
    pl.kernel,
    out_type=jax.ShapeDtypeStruct((N_FLAT, DIM), jnp.float32),
    mesh=plsc.VectorSubcoreMesh(core_axis_name="c", subcore_axis_name="s"),
    scratch_types=[
        pltpu.VMEM((BLOCKS_PER_W, BLOCK), jnp.int32),
        pltpu.VMEM((SEQ, DIM), jnp.float32),
        pltpu.VMEM((NBUF, BLOCK, DIM), jnp.float32),
    ] + [pltpu.SemaphoreType.DMA] * (2 * NBUF),
)
def _emb_sc_kernel(src_hbm, table_hbm, pe_hbm, out_hbm, *scratch):
    _emb_body(src_hbm, table_hbm, pe_hbm, out_hbm, *scratch)


def kernel(src, table):
    idx = src.reshape(NW, BLOCKS_PER_W, BLOCK)
    pe = _positional_encoding_const()
    out = _emb_sc_kernel(idx, table, pe)
    return out.reshape(SEQ, BATCH, DIM)


# speedup vs baseline: 7.4019x; 1.0074x over previous
import functools

import jax
import jax.numpy as jnp
from jax import lax
from jax.experimental import pallas as pl
from jax.experimental.pallas import tpu as pltpu
from jax.experimental.pallas import tpu_sc as plsc

VOCAB = 100000
DIM = 128
SEQ = 200
BATCH = 1024

NC = 2
NS = 16
NW = NC * NS

BLOCK = 64
N_FLAT = SEQ * BATCH
NUM_BLOCKS = N_FLAT // BLOCK
BLOCKS_PER_W = NUM_BLOCKS // NW
BLOCKS_PER_L = BATCH // BLOCK
NBUF = 10
NGROUPS = BLOCKS_PER_W // NBUF
LANES = 16
NVEC = DIM // LANES
ROW_UNROLL = 2


def _emb_body(src_hbm, table_hbm, pe_hbm, out_hbm, idx_v, pe_v, rows_v, *sems):
    gsems = sems[:NBUF]
    ssems = sems[NBUF:]
    c = lax.axis_index("c")
    s = lax.axis_index("s")
    w = s * NC + c

    pltpu.sync_copy(src_hbm.at[w], idx_v)
    pltpu.sync_copy(pe_hbm, pe_v)

    first_blk = w * BLOCKS_PER_W

    def gather_descr(j, b):
        return pltpu.make_async_copy(
            table_hbm.at[idx_v.at[j]], rows_v.at[b], gsems[b])

    def scatter_descr(j, b):
        out_base = (first_blk + j) * BLOCK
        return pltpu.make_async_copy(
            rows_v.at[b], out_hbm.at[pl.ds(out_base, BLOCK)], ssems[b])

    def group(g, carry):
        base = g * NBUF
        for b in range(NBUF):
            j = base + b

            @pl.when(g > 0)
            def _wait_prev_scatter():
                scatter_descr(j - NBUF, b).wait()

            gather_descr(j, b).start()

        for b in range(NBUF):
            j = base + b
            gather_descr(j, b).wait()

            l = (first_blk + j) // BLOCKS_PER_L
            pe_vecs = [pe_v[l, pl.ds(d * LANES, LANES)] for d in range(NVEC)]

            def row_body(r, acc, b=b, pe_vecs=pe_vecs):
                for u in range(ROW_UNROLL):
                    for d in range(NVEC):
                        plsc.addupdate(
                            rows_v.at[b, r * ROW_UNROLL + u,
                                      pl.ds(d * LANES, LANES)], pe_vecs[d])
                return acc

            lax.fori_loop(0, BLOCK // ROW_UNROLL, row_body, 0)
            scatter_descr(j, b).start()
        return carry

    lax.fori_loop(0, NGROUPS, group, 0)
    for b in range(NBUF):
        scatter_descr((NGROUPS - 1) * NBUF + b, b).wait()


def _positional_encoding_const():
    positions = jnp.arange(SEQ, dtype=jnp.float32)[:, None]
    i = jnp.arange(DIM, dtype=jnp.float32)
    div_term = 1.0 / jnp.power(10000.0, (2.0 * i) / DIM)
    angles = positions * div_term[None, :]
    even_mask = (jnp.arange(DIM) % 2 == 0)
    return jnp.where(even_mask[None, :], jnp.sin(angles), jnp.cos(angles))


@functools.partial(
    pl.kernel,
    out_type=jax.ShapeDtypeStruct((N_FLAT, DIM), jnp.float32),
    mesh=plsc.VectorSubcoreMesh(core_axis_name="c", subcore_axis_name="s"),
    scratch_types=[
        pltpu.VMEM((BLOCKS_PER_W, BLOCK), jnp.int32),
        pltpu.VMEM((SEQ, DIM), jnp.float32),
        pltpu.VMEM((NBUF, BLOCK, DIM), jnp.float32),
    ] + [pltpu.SemaphoreType.DMA] * (2 * NBUF),
)
def _emb_sc_kernel(src_hbm, table_hbm, pe_hbm, out_hbm, *scratch):
    _emb_body(src_hbm, table_hbm, pe_hbm, out_hbm, *scratch)


def kernel(src, table):
    idx = src.reshape(NW, BLOCKS_PER_W, BLOCK)
    pe = _positional_encoding_const()
    out = _emb_sc_kernel(idx, table, pe)
    return out.reshape(SEQ, BATCH, DIM)
